# Initial kernel scaffold; baseline (speedup 1.0000x reference)
import jax
import jax.numpy as jnp
from jax.experimental import pallas as pl


def kernel(feature, edge_index, W0, b0, W1, b1, W_prompt, W_pp):
    raise NotImplementedError("write your pallas kernel here")


# speedup vs baseline: 3.0109x; 3.0109x over previous
import functools

import jax
import jax.numpy as jnp
from jax import lax
from jax.experimental import pallas as pl
from jax.experimental.pallas import tpu as pltpu
from jax.experimental.pallas import tpu_sc as plsc

_N = 10000
_E = 160000
_IN = 256
_H = 512
_NC = 64
_CENTERS = 7

_LANE = 128
_CHUNK = 128
_NCORE = 2
_NSUB = 16
_CPT = 80
_EPAD = _NSUB * _CPT * _CHUNK
_NP = 10240
_ACC_ROWS = _NP
_ZROWS = _ACC_ROWS // _NSUB
_OROWS = _ZROWS

_BN = 400
_NBLK = _N // _BN


def _make_segment_sum(nb):
    nper = nb // _NCORE
    mesh = plsc.VectorSubcoreMesh(core_axis_name="c", subcore_axis_name="s")

    def body(xr, srcm, dstm, zeros, out, acc, src_v, dst_v, rows_v, sem):
        c = lax.axis_index("c")
        s = lax.axis_index("s")
        pltpu.sync_copy(srcm.at[pl.ds(s * _CPT, _CPT)], src_v)
        pltpu.sync_copy(dstm.at[pl.ds(s * _CPT, _CPT)], dst_v)
        for p in range(nper):
            b = c * nper + p
            pltpu.sync_copy(zeros, acc.at[pl.ds(s * _ZROWS, _ZROWS)])
            plsc.subcore_barrier()

            def chunk(j, carry):
                pltpu.async_copy(xr.at[b].at[src_v.at[j]], rows_v, sem).wait()
                pltpu.sync_copy(rows_v, acc.at[dst_v.at[j]], add=True)
                return carry

            lax.fori_loop(0, _CPT, chunk, 0)
            plsc.subcore_barrier()
            pltpu.sync_copy(acc.at[pl.ds(s * _OROWS, _OROWS)],
                            out.at[b].at[pl.ds(s * _OROWS, _OROWS)])
            plsc.subcore_barrier()

    return pl.kernel(
        body,
        out_type=jax.ShapeDtypeStruct((nb, _NP, _LANE), jnp.float32),
        mesh=mesh,
        scratch_types=[
            pltpu.VMEM_SHARED((_ACC_ROWS, _LANE), jnp.float32),
            pltpu.VMEM((_CPT, _CHUNK), jnp.int32),
            pltpu.VMEM((_CPT, _CHUNK), jnp.int32),
            pltpu.VMEM((_CHUNK, _LANE), jnp.float32),
            pltpu.SemaphoreType.DMA,
        ],
    )


@functools.cache
def _get_seg(nb):
    return _make_segment_sum(nb)


def _mm0_body(a_ref, w_ref, b_ref, o_ref):
    w = w_ref[...]
    acc = jnp.dot(a_ref[0], w[:_LANE, :], preferred_element_type=jnp.float32)
    acc = acc + jnp.dot(a_ref[1], w[_LANE:, :], preferred_element_type=jnp.float32)
    h = jnp.maximum(acc + b_ref[...], 0.0)
    for k in range(4):
        o_ref[k] = h[:, k * _LANE:(k + 1) * _LANE]


_mm0 = pl.pallas_call(
    _mm0_body,
    grid=(_NBLK,),
    in_specs=[
        pl.BlockSpec((2, _BN, _LANE), lambda i: (0, i, 0)),
        pl.BlockSpec((_IN, _H), lambda i: (0, 0)),
        pl.BlockSpec((1, _H), lambda i: (0, 0)),
    ],
    out_specs=pl.BlockSpec((4, _BN, _LANE), lambda i: (0, i, 0)),
    out_shape=jax.ShapeDtypeStruct((4, _NP, _LANE), jnp.float32),
)


def _dotT(x, w):
    return lax.dot_general(x, w, (((1,), (1,)), ((), ())),
                           preferred_element_type=jnp.float32)


def _final_body(a_ref, h1_ref, w1_ref, b1_ref, wp_ref, wpp_ref, o_ref):
    h2 = jnp.dot(a_ref[0], w1_ref[0], preferred_element_type=jnp.float32)
    for k in range(1, 4):
        h2 = h2 + jnp.dot(a_ref[k], w1_ref[k], preferred_element_type=jnp.float32)
    h2 = jnp.maximum(h2 + b1_ref[...], 0.0)

    wp = wp_ref[...]
    lg = _dotT(h2, wp[:, :_H])
    for k in range(4):
        lg = lg + _dotT(h1_ref[k], wp[:, _H + k * _LANE:_H + (k + 1) * _LANE])
    colid = lax.broadcasted_iota(jnp.int32, lg.shape, 1)
    lg = jnp.where(colid < _CENTERS, lg, -jnp.inf)
    m = jnp.max(lg, axis=1, keepdims=True)
    idx = jnp.min(jnp.where(lg == m, colid, _CENTERS), axis=1, keepdims=True)

    acc = jnp.zeros((_BN, _NC), jnp.float32)
    for c in range(_CENTERS):
        wc = wpp_ref[c]
        oc = _dotT(h2, wc[:, :_H])
        for k in range(4):
            oc = oc + _dotT(h1_ref[k], wc[:, _H + k * _LANE:_H + (k + 1) * _LANE])
        acc = acc + jnp.where(idx == c, oc, 0.0)
    o_ref[...] = acc


_final = pl.pallas_call(
    _final_body,
    grid=(_NBLK,),
    in_specs=[
        pl.BlockSpec((4, _BN, _LANE), lambda i: (0, i, 0)),
        pl.BlockSpec((4, _BN, _LANE), lambda i: (0, i, 0)),
        pl.BlockSpec((4, _LANE, _H), lambda i: (0, 0, 0)),
        pl.BlockSpec((1, _H), lambda i: (0, 0)),
        pl.BlockSpec((8, 2 * _H), lambda i: (0, 0)),
        pl.BlockSpec((_CENTERS, _NC, 2 * _H), lambda i: (0, 0, 0)),
    ],
    out_specs=pl.BlockSpec((_BN, _NC), lambda i: (i, 0)),
    out_shape=jax.ShapeDtypeStruct((_N, _NC), jnp.float32),
)


def kernel(feature, edge_index, W0, b0, W1, b1, W_prompt, W_pp):
    src = edge_index[0]
    dst = edge_index[1]
    pad = _EPAD - _E
    srcm = jnp.concatenate([src, jnp.zeros((pad,), jnp.int32)]).reshape(-1, _CHUNK)
    dstm = jnp.concatenate([dst, jnp.full((pad,), _N, jnp.int32)]).reshape(-1, _CHUNK)
    zeros = jnp.zeros((_ZROWS, _LANE), jnp.float32)

    featT = feature.reshape(_N, 2, _LANE).transpose(1, 0, 2)
    featT = jnp.pad(featT, ((0, 0), (0, _NP - _N), (0, 0)))
    agg0 = _get_seg(2)(featT, srcm, dstm, zeros)
    h1 = _mm0(agg0, W0, b0.reshape(1, _H))
    agg1 = _get_seg(4)(h1, srcm, dstm, zeros)
    wp_pad = jnp.concatenate(
        [W_prompt, jnp.zeros((1, 2 * _H), jnp.float32)], axis=0)
    out = _final(agg1, h1, W1.reshape(4, _LANE, _H), b1.reshape(1, _H),
                 wp_pad, W_pp)
    return out


# speedup vs baseline: 3.3413x; 1.1097x over previous
import functools

import jax
import jax.numpy as jnp
from jax import lax
from jax.experimental import pallas as pl
from jax.experimental.pallas import tpu as pltpu
from jax.experimental.pallas import tpu_sc as plsc

_N = 10000
_E = 160000
_IN = 256
_H = 512
_NC = 64
_CENTERS = 7

_LANE = 128
_CHUNK = 128
_NCORE = 2
_NSUB = 16
_CPT = 80
_EPAD = _NSUB * _CPT * _CHUNK
_NP = 10240
_ACC_ROWS = _NP
_ZROWS = _ACC_ROWS // _NSUB
_OROWS = _ZROWS

_NBUF = 2
_HALF = _CPT // 2

_BN = 400
_NBLK = _N // _BN


def _make_segment_sum(nb):
    nper = nb // _NCORE
    mesh = plsc.VectorSubcoreMesh(core_axis_name="c", subcore_axis_name="s")

    def body(xr, srcm, dstm, zeros, out, acc, src_v, dst_v, rows_v, g0, g1,
             s0, s1):
        gsem = (g0, g1)
        ssem = (s0, s1)
        c = lax.axis_index("c")
        s = lax.axis_index("s")
        for p in range(nper):
            b = c * nper + p
            pltpu.sync_copy(zeros, acc.at[pl.ds(s * _ZROWS, _ZROWS)])
            plsc.subcore_barrier()

            for h in range(2):
                pltpu.sync_copy(
                    srcm.at[pl.ds(s * _CPT + h * _HALF, _HALF)], src_v)
                pltpu.sync_copy(
                    dstm.at[pl.ds(s * _CPT + h * _HALF, _HALF)], dst_v)

                def group(i, carry):
                    jb = i * _NBUF
                    for k in range(_NBUF):
                        @pl.when(i > 0)
                        def _drain(k=k, jb=jb):
                            pltpu.make_async_copy(
                                rows_v.at[k],
                                acc.at[dst_v.at[jb - _NBUF + k]],
                                ssem[k]).wait()
                        pltpu.async_copy(xr.at[b].at[src_v.at[jb + k]],
                                         rows_v.at[k], gsem[k])
                    for k in range(_NBUF):
                        pltpu.make_async_copy(xr.at[b].at[src_v.at[jb + k]],
                                              rows_v.at[k], gsem[k]).wait()
                        pltpu.async_copy(rows_v.at[k],
                                         acc.at[dst_v.at[jb + k]],
                                         ssem[k], add=True)
                    return carry

                lax.fori_loop(0, _HALF // _NBUF, group, 0)
                for k in range(_NBUF):
                    pltpu.make_async_copy(
                        rows_v.at[k], acc.at[dst_v.at[_HALF - _NBUF + k]],
                        ssem[k]).wait()
            plsc.subcore_barrier()
            pltpu.sync_copy(acc.at[pl.ds(s * _OROWS, _OROWS)],
                            out.at[b].at[pl.ds(s * _OROWS, _OROWS)])
            plsc.subcore_barrier()

    return pl.kernel(
        body,
        out_type=jax.ShapeDtypeStruct((nb, _NP, _LANE), jnp.float32),
        mesh=mesh,
        scratch_types=[
            pltpu.VMEM_SHARED((_ACC_ROWS, _LANE), jnp.float32),
            pltpu.VMEM((_HALF, _CHUNK), jnp.int32),
            pltpu.VMEM((_HALF, _CHUNK), jnp.int32),
            pltpu.VMEM((_NBUF, _CHUNK, _LANE), jnp.float32),
        ] + [pltpu.SemaphoreType.DMA] * (2 * _NBUF),
    )


@functools.cache
def _get_seg(nb):
    return _make_segment_sum(nb)


def _mm0_body(a_ref, w_ref, b_ref, o_ref):
    w = w_ref[...]
    acc = jnp.dot(a_ref[0], w[:_LANE, :], preferred_element_type=jnp.float32)
    acc = acc + jnp.dot(a_ref[1], w[_LANE:, :], preferred_element_type=jnp.float32)
    h = jnp.maximum(acc + b_ref[...], 0.0)
    for k in range(4):
        o_ref[k] = h[:, k * _LANE:(k + 1) * _LANE]


_mm0 = pl.pallas_call(
    _mm0_body,
    grid=(_NBLK,),
    in_specs=[
        pl.BlockSpec((2, _BN, _LANE), lambda i: (0, i, 0)),
        pl.BlockSpec((_IN, _H), lambda i: (0, 0)),
        pl.BlockSpec((1, _H), lambda i: (0, 0)),
    ],
    out_specs=pl.BlockSpec((4, _BN, _LANE), lambda i: (0, i, 0)),
    out_shape=jax.ShapeDtypeStruct((4, _NP, _LANE), jnp.float32),
)


def _dotT(x, w):
    return lax.dot_general(x, w, (((1,), (1,)), ((), ())),
                           preferred_element_type=jnp.float32)


def _final_body(a_ref, h1_ref, w1_ref, b1_ref, wp_ref, wpp_ref, o_ref):
    h2 = jnp.dot(a_ref[0], w1_ref[0], preferred_element_type=jnp.float32)
    for k in range(1, 4):
        h2 = h2 + jnp.dot(a_ref[k], w1_ref[k], preferred_element_type=jnp.float32)
    h2 = jnp.maximum(h2 + b1_ref[...], 0.0)

    wp = wp_ref[...]
    lg = _dotT(h2, wp[:, :_H])
    for k in range(4):
        lg = lg + _dotT(h1_ref[k], wp[:, _H + k * _LANE:_H + (k + 1) * _LANE])
    colid = lax.broadcasted_iota(jnp.int32, lg.shape, 1)
    lg = jnp.where(colid < _CENTERS, lg, -jnp.inf)
    m = jnp.max(lg, axis=1, keepdims=True)
    idx = jnp.min(jnp.where(lg == m, colid, _CENTERS), axis=1, keepdims=True)

    acc = jnp.zeros((_BN, _NC), jnp.float32)
    for c in range(_CENTERS):
        wc = wpp_ref[c]
        oc = _dotT(h2, wc[:, :_H])
        for k in range(4):
            oc = oc + _dotT(h1_ref[k], wc[:, _H + k * _LANE:_H + (k + 1) * _LANE])
        acc = acc + jnp.where(idx == c, oc, 0.0)
    o_ref[...] = acc


_final = pl.pallas_call(
    _final_body,
    grid=(_NBLK,),
    in_specs=[
        pl.BlockSpec((4, _BN, _LANE), lambda i: (0, i, 0)),
        pl.BlockSpec((4, _BN, _LANE), lambda i: (0, i, 0)),
        pl.BlockSpec((4, _LANE, _H), lambda i: (0, 0, 0)),
        pl.BlockSpec((1, _H), lambda i: (0, 0)),
        pl.BlockSpec((8, 2 * _H), lambda i: (0, 0)),
        pl.BlockSpec((_CENTERS, _NC, 2 * _H), lambda i: (0, 0, 0)),
    ],
    out_specs=pl.BlockSpec((_BN, _NC), lambda i: (i, 0)),
    out_shape=jax.ShapeDtypeStruct((_N, _NC), jnp.float32),
)


def kernel(feature, edge_index, W0, b0, W1, b1, W_prompt, W_pp):
    src = edge_index[0]
    dst = edge_index[1]
    pad = _EPAD - _E
    srcm = jnp.concatenate([src, jnp.zeros((pad,), jnp.int32)]).reshape(-1, _CHUNK)
    dstm = jnp.concatenate([dst, jnp.full((pad,), _N, jnp.int32)]).reshape(-1, _CHUNK)
    zeros = jnp.zeros((_ZROWS, _LANE), jnp.float32)

    featT = feature.reshape(_N, 2, _LANE).transpose(1, 0, 2)
    featT = jnp.pad(featT, ((0, 0), (0, _NP - _N), (0, 0)))
    agg0 = _get_seg(2)(featT, srcm, dstm, zeros)
    h1 = _mm0(agg0, W0, b0.reshape(1, _H))
    agg1 = _get_seg(4)(h1, srcm, dstm, zeros)
    wp_pad = jnp.concatenate(
        [W_prompt, jnp.zeros((1, 2 * _H), jnp.float32)], axis=0)
    out = _final(agg1, h1, W1.reshape(4, _LANE, _H), b1.reshape(1, _H),
                 wp_pad, W_pp)
    return out
